# baseline (speedup 1.0000x reference)
import functools

import jax
import jax.numpy as jnp
from jax import lax
from jax.experimental import pallas as pl
from jax.experimental.pallas import tpu as pltpu
from jax.experimental.pallas import tpu_sc as plsc

B = 32
IN_SCALE = 7
D = IN_SCALE * IN_SCALE
BITS = 32
HDC = 512
NNZ = D * HDC // 2
IND_ROWS = 64
IND_PAD = IND_ROWS * HDC
LANES = 16
NUM_CORES = 2
NUM_SUBCORES = 16
NW = NUM_CORES * NUM_SUBCORES
CHUNK = IND_PAD // NW
CROWS = CHUNK // HDC


def _sc_scatter_body(idx_hbm, out_hbm, idx_v, chunk_v):
    wid = lax.axis_index("s") * NUM_CORES + lax.axis_index("c")
    base = wid * CHUNK

    z16 = jnp.zeros((LANES,), jnp.int32)

    def zero_body(i, carry):
        chunk_v[i // (HDC // LANES), pl.ds((i % (HDC // LANES)) * LANES, LANES)] = z16
        return carry

    lax.fori_loop(0, CHUNK // LANES, zero_body, 0)

    pltpu.sync_copy(idx_hbm, idx_v)

    ones16 = jnp.ones((LANES,), jnp.int32)

    def scat_body(i, carry):
        v = idx_v[pl.ds(i * LANES, LANES)]
        local = v - base
        m = (local >= 0) & (local < CHUNK)
        lc = jnp.clip(local, 0, CHUNK - 1)
        plsc.store_scatter(chunk_v, [lc >> 9, lc & (HDC - 1)], ones16, mask=m)
        return carry

    lax.fori_loop(0, NNZ // LANES, scat_body, 0)

    pltpu.sync_copy(chunk_v, out_hbm.at[pl.ds(wid * CROWS, CROWS), :])


@functools.cache
def _make_sc_scatter():
    return functools.partial(
        pl.kernel,
        mesh=plsc.VectorSubcoreMesh(
            core_axis_name="c", subcore_axis_name="s",
            num_cores=NUM_CORES, num_subcores=NUM_SUBCORES,
        ),
        out_type=jax.ShapeDtypeStruct((IND_ROWS, HDC), jnp.int32),
        scratch_types=[
            pltpu.VMEM((NNZ,), jnp.int32),
            pltpu.VMEM((CROWS, HDC), jnp.int32),
        ],
        compiler_params=pltpu.CompilerParams(needs_layout_passes=False),
    )(_sc_scatter_body)


def _tc_main_body(xi_ref, di_ref, di4_ref, st_ref, st3_ref, par_ref, par3_ref,
                  rf_ref, gf_ref, bf_ref, af_ref, sf_ref,
                  r_ref, g_ref, b_ref, a_ref, s_ref,
                  proj_ref, rbin_ref):
    blk = pl.program_id(0)

    @pl.when(blk == 0)
    def _():
        sf_ref[...] = st_ref[...] * par_ref[...]
        s_ref[...] = st3_ref[...] * par3_ref[...]
        rf_ref[...] = di4_ref[:, 0, :]
        gf_ref[...] = di4_ref[:, 1, :]
        bf_ref[...] = di4_ref[:, 2, :]
        af_ref[...] = di4_ref[:, 3, :]
        r_ref[...] = di_ref[:, 0 * IN_SCALE:1 * IN_SCALE, :]
        g_ref[...] = di_ref[:, 1 * IN_SCALE:2 * IN_SCALE, :]
        b_ref[...] = di_ref[:, 2 * IN_SCALE:3 * IN_SCALE, :]
        a_ref[...] = di_ref[:, 3 * IN_SCALE:4 * IN_SCALE, :]

    bit_iota = lax.broadcasted_iota(jnp.int32, (BITS, HDC), 0)
    ztile = jnp.zeros((BITS, HDC), jnp.int32)

    for j in range(D):
        x = xi_ref[0, 0, 0, j]
        xb = jnp.bitwise_and(jnp.right_shift(x, bit_iota), 1)
        proj_ref[0, j] = ztile
        rbin_ref[0, j] = xb


def _const0(b):
    return (0, 0)


def _const3(b):
    return (0, 0, 0)


_tc_main = pl.pallas_call(
    _tc_main_body,
    grid=(B,),
    in_specs=[
        pl.BlockSpec((1, 1, 1, D), lambda b: (b, 0, 0, 0),
                     memory_space=pltpu.SMEM),
        pl.BlockSpec((B, 4 * IN_SCALE, IN_SCALE), _const3),
        pl.BlockSpec((B, 4, D), _const3),
        pl.BlockSpec((B, D), _const0),
        pl.BlockSpec((B, IN_SCALE, IN_SCALE), _const3),
        pl.BlockSpec((1, D), _const0),
        pl.BlockSpec((1, IN_SCALE, IN_SCALE), _const3),
    ],
    out_specs=[
        pl.BlockSpec((B, D), _const0),
        pl.BlockSpec((B, D), _const0),
        pl.BlockSpec((B, D), _const0),
        pl.BlockSpec((B, D), _const0),
        pl.BlockSpec((B, D), _const0),
        pl.BlockSpec((B, IN_SCALE, IN_SCALE), _const3),
        pl.BlockSpec((B, IN_SCALE, IN_SCALE), _const3),
        pl.BlockSpec((B, IN_SCALE, IN_SCALE), _const3),
        pl.BlockSpec((B, IN_SCALE, IN_SCALE), _const3),
        pl.BlockSpec((B, IN_SCALE, IN_SCALE), _const3),
        pl.BlockSpec((1, D, BITS, HDC), lambda b: (b, 0, 0, 0)),
        pl.BlockSpec((1, D, BITS, HDC), lambda b: (b, 0, 0, 0)),
    ],
    out_shape=[
        jax.ShapeDtypeStruct((B, D), jnp.float32),
        jax.ShapeDtypeStruct((B, D), jnp.float32),
        jax.ShapeDtypeStruct((B, D), jnp.float32),
        jax.ShapeDtypeStruct((B, D), jnp.float32),
        jax.ShapeDtypeStruct((B, D), jnp.float32),
        jax.ShapeDtypeStruct((B, IN_SCALE, IN_SCALE), jnp.float32),
        jax.ShapeDtypeStruct((B, IN_SCALE, IN_SCALE), jnp.float32),
        jax.ShapeDtypeStruct((B, IN_SCALE, IN_SCALE), jnp.float32),
        jax.ShapeDtypeStruct((B, IN_SCALE, IN_SCALE), jnp.float32),
        jax.ShapeDtypeStruct((B, IN_SCALE, IN_SCALE), jnp.float32),
        jax.ShapeDtypeStruct((B, D, BITS, HDC), jnp.int32),
        jax.ShapeDtypeStruct((B, D, BITS, HDC), jnp.int32),
    ],
)


def _tc_patch_body(ind_ref, pin_ref, rin_ref, pout_ref, rout_ref):
    del pin_ref
    for j in range(2):
        tile = ind_ref[j * BITS:(j + 1) * BITS, :]
        pout_ref[0, j] = tile
        rout_ref[0, j] = jnp.bitwise_xor(rin_ref[0, j], tile)


_tc_patch = pl.pallas_call(
    _tc_patch_body,
    grid=(1,),
    in_specs=[
        pl.BlockSpec((IND_ROWS, HDC), lambda i: (0, 0)),
        pl.BlockSpec((1, 1, 8, 128), lambda i: (0, 0, 0, 0)),
        pl.BlockSpec((1, 2, BITS, HDC), lambda i: (0, 0, 0, 0)),
    ],
    out_specs=[
        pl.BlockSpec((1, 2, BITS, HDC), lambda i: (0, 0, 0, 0)),
        pl.BlockSpec((1, 2, BITS, HDC), lambda i: (0, 0, 0, 0)),
    ],
    out_shape=[
        jax.ShapeDtypeStruct((B, D, BITS, HDC), jnp.int32),
        jax.ShapeDtypeStruct((B, D, BITS, HDC), jnp.int32),
    ],
    input_output_aliases={1: 0, 2: 1},
)


def kernel(data_input, structure_input, meta_input_h1, meta_input_h2,
           meta_input_h3, meta_input_h4, meta_input_h5, noise_var_in_binary,
           fmot_in_binary, meta_output_h1, meta_output_h2, meta_output_h3,
           meta_output_h4, meta_output_h5, noise_var_out, non_zero_indices,
           parameters_temp):
    di4 = data_input.reshape(B, 4, D)
    rf0 = di4[:, 0, :]
    xi = lax.bitcast_convert_type(rf0, jnp.int32).reshape(B, 1, 1, D)
    st = structure_input.reshape(B, D)
    par = parameters_temp.reshape(1, D)
    par3 = parameters_temp.reshape(1, IN_SCALE, IN_SCALE)

    ind2d = _make_sc_scatter()(non_zero_indices)

    rf, gf, bf, af, sf, r, g, bch, a, s, proj0, rbin0 = _tc_main(
        xi, data_input, di4, st, structure_input, par, par3)
    proj, r_bin = _tc_patch(ind2d, proj0, rbin0)

    deepS = (r, g, bch, a, s)
    return (rf, gf, bf, af, sf, deepS, proj, r_bin)


# speedup vs baseline: 1.1115x; 1.1115x over previous
import functools

import jax
import jax.numpy as jnp
from jax import lax
from jax.experimental import pallas as pl
from jax.experimental.pallas import tpu as pltpu
from jax.experimental.pallas import tpu_sc as plsc

B = 32
IN_SCALE = 7
D = IN_SCALE * IN_SCALE
BITS = 32
HDC = 512
NNZ = D * HDC // 2
IND_ROWS = 64
IND_PAD = IND_ROWS * HDC
LANES = 16
NUM_CORES = 2
NUM_SUBCORES = 16
NW = NUM_CORES * NUM_SUBCORES
CHUNK = IND_PAD // NW
CROWS = CHUNK // HDC
XROWS = 8


def _sc_scatter_body(idx_hbm, out_hbm, idx_v, chunk_v):
    wid = lax.axis_index("s") * NUM_CORES + lax.axis_index("c")
    base = wid * CHUNK

    z16 = jnp.zeros((LANES,), jnp.int32)

    def zero_body(i, carry):
        chunk_v[i // (HDC // LANES), pl.ds((i % (HDC // LANES)) * LANES, LANES)] = z16
        return carry

    lax.fori_loop(0, CHUNK // LANES, zero_body, 0)

    pltpu.sync_copy(idx_hbm, idx_v)

    ones16 = jnp.ones((LANES,), jnp.int32)

    def scat_body(i, carry):
        v = idx_v[pl.ds(i * LANES, LANES)]
        local = v - base
        m = (local >= 0) & (local < CHUNK)
        lc = jnp.clip(local, 0, CHUNK - 1)
        plsc.store_scatter(chunk_v, [lc >> 9, lc & (HDC - 1)], ones16, mask=m)
        return carry

    lax.fori_loop(0, NNZ // LANES, scat_body, 0)

    pltpu.sync_copy(chunk_v, out_hbm.at[pl.ds(wid * CROWS, CROWS), :])


@functools.cache
def _make_sc_scatter():
    return functools.partial(
        pl.kernel,
        mesh=plsc.VectorSubcoreMesh(
            core_axis_name="c", subcore_axis_name="s",
            num_cores=NUM_CORES, num_subcores=NUM_SUBCORES,
        ),
        out_type=jax.ShapeDtypeStruct((IND_ROWS, HDC), jnp.int32),
        scratch_types=[
            pltpu.VMEM((NNZ,), jnp.int32),
            pltpu.VMEM((CROWS, HDC), jnp.int32),
        ],
        compiler_params=pltpu.CompilerParams(needs_layout_passes=False),
    )(_sc_scatter_body)


def _tc_main_body(xi_ref, st_ref, par_ref, sf_ref, proj_ref, rbin_ref):
    blk = pl.program_id(0)

    @pl.when(blk == 0)
    def _():
        sf_ref[...] = st_ref[...] * par_ref[...]

    bit_iota = lax.broadcasted_iota(jnp.int32, (BITS, HDC), 0)
    ztile = jnp.zeros((BITS, HDC), jnp.int32)

    row = blk % XROWS
    for j in range(D):
        x = xi_ref[row, j]
        xb = jnp.bitwise_and(jnp.right_shift(x, bit_iota), 1)
        proj_ref[0, j] = ztile
        rbin_ref[0, j] = xb


_tc_main = pl.pallas_call(
    _tc_main_body,
    grid=(B,),
    in_specs=[
        pl.BlockSpec((XROWS, D), lambda b: (b // XROWS, 0),
                     memory_space=pltpu.SMEM),
        pl.BlockSpec((B, D), lambda b: (0, 0)),
        pl.BlockSpec((1, D), lambda b: (0, 0)),
    ],
    out_specs=[
        pl.BlockSpec((B, D), lambda b: (0, 0)),
        pl.BlockSpec((1, D, BITS, HDC), lambda b: (b, 0, 0, 0)),
        pl.BlockSpec((1, D, BITS, HDC), lambda b: (b, 0, 0, 0)),
    ],
    out_shape=[
        jax.ShapeDtypeStruct((B, D), jnp.float32),
        jax.ShapeDtypeStruct((B, D, BITS, HDC), jnp.int32),
        jax.ShapeDtypeStruct((B, D, BITS, HDC), jnp.int32),
    ],
)


def _tc_patch_body(ind_ref, pin_ref, rin_ref, pout_ref, rout_ref):
    del pin_ref
    for j in range(2):
        tile = ind_ref[j * BITS:(j + 1) * BITS, :]
        pout_ref[0, j] = tile
        rout_ref[0, j] = jnp.bitwise_xor(rin_ref[0, j], tile)


_tc_patch = pl.pallas_call(
    _tc_patch_body,
    grid=(1,),
    in_specs=[
        pl.BlockSpec((IND_ROWS, HDC), lambda i: (0, 0)),
        pl.BlockSpec((1, 1, 8, 128), lambda i: (0, 0, 0, 0)),
        pl.BlockSpec((1, 2, BITS, HDC), lambda i: (0, 0, 0, 0)),
    ],
    out_specs=[
        pl.BlockSpec((1, 2, BITS, HDC), lambda i: (0, 0, 0, 0)),
        pl.BlockSpec((1, 2, BITS, HDC), lambda i: (0, 0, 0, 0)),
    ],
    out_shape=[
        jax.ShapeDtypeStruct((B, D, BITS, HDC), jnp.int32),
        jax.ShapeDtypeStruct((B, D, BITS, HDC), jnp.int32),
    ],
    input_output_aliases={1: 0, 2: 1},
)


def kernel(data_input, structure_input, meta_input_h1, meta_input_h2,
           meta_input_h3, meta_input_h4, meta_input_h5, noise_var_in_binary,
           fmot_in_binary, meta_output_h1, meta_output_h2, meta_output_h3,
           meta_output_h4, meta_output_h5, noise_var_out, non_zero_indices,
           parameters_temp):
    r = data_input[:, 0:IN_SCALE, :]
    g = data_input[:, IN_SCALE:2 * IN_SCALE, :]
    bch = data_input[:, 2 * IN_SCALE:3 * IN_SCALE, :]
    a = data_input[:, 3 * IN_SCALE:4 * IN_SCALE, :]
    rf = r.reshape(B, D)
    gf = g.reshape(B, D)
    bf = bch.reshape(B, D)
    af = a.reshape(B, D)

    xi = lax.bitcast_convert_type(rf, jnp.int32)
    st = structure_input.reshape(B, D)
    par = parameters_temp.reshape(1, D)

    ind2d = _make_sc_scatter()(non_zero_indices)

    sf, proj0, rbin0 = _tc_main(xi, st, par)
    proj, r_bin = _tc_patch(ind2d, proj0, rbin0)

    s = sf.reshape(B, IN_SCALE, IN_SCALE)
    deepS = (r, g, bch, a, s)
    return (rf, gf, bf, af, sf, deepS, proj, r_bin)
